# baseline (speedup 1.0000x reference)
import functools

import jax
import jax.numpy as jnp
from jax.experimental import pallas as pl
from jax.experimental.pallas import tpu as pltpu

_HIDDEN = 4096
_EXPERTS = 128
_TOPK = 8
_TOKENS = 8192
_TB = 1024
_SUB = 128


def _router_block(hs_ref, w_ref, b_ref, idx_ref, wgt_ref, sc_ref):
    logits = jnp.dot(hs_ref[...], w_ref[...], preferred_element_type=jnp.float32)
    sc_ref[...] = jax.nn.sigmoid(logits)
    brow = b_ref[...]
    lanef = jax.lax.broadcasted_iota(jnp.int32, (_SUB, _EXPERTS), 1).astype(
        jnp.float32
    )
    crow = lanef + brow

    def _tile(t, carry):
        base = t * _SUB
        vals = sc_ref[pl.ds(base, _SUB), :] + brow
        m_cols = []
        c_cols = []
        for _ in range(_TOPK):
            m = jnp.max(vals, axis=1, keepdims=True)
            eq = vals == m
            c = jnp.min(jnp.where(eq, crow, jnp.inf), axis=1, keepdims=True)
            vals = jnp.where(crow == c, -jnp.inf, vals)
            m_cols.append(m)
            c_cols.append(c)
        mcat = jnp.concatenate(m_cols, axis=1)
        ccat = jnp.concatenate(c_cols, axis=1)
        idxf = jnp.floor(ccat + 0.5)
        ws = mcat - (ccat - idxf)
        ws = ws / (jnp.sum(ws, axis=1, keepdims=True) + 1e-20)
        idx_ref[pl.ds(base, _SUB), :] = idxf.astype(jnp.int32)
        wgt_ref[pl.ds(base, _SUB), :] = ws
        return carry

    jax.lax.fori_loop(0, _TB // _SUB, _tile, 0)


@functools.partial(jax.jit)
def kernel(hidden_states, weight, e_score_correction_bias):
    hs = hidden_states.reshape(-1, _HIDDEN)
    wt = weight.astype(jnp.float32).T
    bias = e_score_correction_bias.reshape(1, _EXPERTS)
    grid = (_TOKENS // _TB,)
    idxs, ws = pl.pallas_call(
        _router_block,
        grid=grid,
        in_specs=[
            pl.BlockSpec((_TB, _HIDDEN), lambda i: (i, 0)),
            pl.BlockSpec((_HIDDEN, _EXPERTS), lambda i: (0, 0)),
            pl.BlockSpec((1, _EXPERTS), lambda i: (0, 0)),
        ],
        out_specs=[
            pl.BlockSpec((_TB, _TOPK), lambda i: (i, 0)),
            pl.BlockSpec((_TB, _TOPK), lambda i: (i, 0)),
        ],
        out_shape=[
            jax.ShapeDtypeStruct((_TOKENS, _TOPK), jnp.int32),
            jax.ShapeDtypeStruct((_TOKENS, _TOPK), jnp.float32),
        ],
        scratch_shapes=[pltpu.VMEM((_TB, _EXPERTS), jnp.float32)],
    )(hs, wt, bias)
    return idxs, ws


# speedup vs baseline: 1.9014x; 1.9014x over previous
import functools

import jax
import jax.numpy as jnp
from jax.experimental import pallas as pl
from jax.experimental.pallas import tpu as pltpu

_HIDDEN = 4096
_EXPERTS = 128
_TOPK = 8
_TOKENS = 8192
_TB = 1024
_NB = _TOKENS // _TB


def _topk_from(sc_ref, b_ref, idx_ref, wgt_ref):
    vals = sc_ref[...] + b_ref[...]
    lanef = jax.lax.broadcasted_iota(jnp.int32, (_TB, _EXPERTS), 1).astype(
        jnp.float32
    )
    crow = lanef + b_ref[...]
    m_cols = []
    c_cols = []
    for _ in range(_TOPK):
        m = jnp.max(vals, axis=1, keepdims=True)
        eq = vals == m
        c = jnp.min(jnp.where(eq, crow, jnp.inf), axis=1, keepdims=True)
        vals = jnp.where(crow == c, -jnp.inf, vals)
        m_cols.append(m)
        c_cols.append(c)
    mcat = jnp.concatenate(m_cols, axis=1)
    ccat = jnp.concatenate(c_cols, axis=1)
    idxf = jnp.floor(ccat + 0.5)
    ws = mcat - (ccat - idxf)
    ws = ws / (jnp.sum(ws, axis=1, keepdims=True) + 1e-20)
    idx_ref[...] = idxf.astype(jnp.int32)
    wgt_ref[...] = ws


def _router_block(hs_ref, w_ref, b_ref, idx_ref, wgt_ref, scA, scB):
    i = pl.program_id(0)

    @pl.when(i < _NB)
    def _mm():
        logits = jnp.dot(
            hs_ref[...], w_ref[...], preferred_element_type=jnp.float32
        )
        s = jax.nn.sigmoid(logits)

        @pl.when(i % 2 == 0)
        def _sa():
            scA[...] = s

        @pl.when(i % 2 == 1)
        def _sb():
            scB[...] = s

    @pl.when(i > 0)
    def _tk():
        @pl.when(i % 2 == 1)
        def _ta():
            _topk_from(scA, b_ref, idx_ref, wgt_ref)

        @pl.when(i % 2 == 0)
        def _tb():
            _topk_from(scB, b_ref, idx_ref, wgt_ref)


@functools.partial(jax.jit)
def kernel(hidden_states, weight, e_score_correction_bias):
    hs = hidden_states.reshape(-1, _HIDDEN)
    wt = weight.astype(jnp.float32).T
    bias = e_score_correction_bias.reshape(1, _EXPERTS)
    grid = (_NB + 1,)
    idxs, ws = pl.pallas_call(
        _router_block,
        grid=grid,
        in_specs=[
            pl.BlockSpec((_TB, _HIDDEN), lambda i: (jnp.minimum(i, _NB - 1), 0)),
            pl.BlockSpec((_HIDDEN, _EXPERTS), lambda i: (0, 0)),
            pl.BlockSpec((1, _EXPERTS), lambda i: (0, 0)),
        ],
        out_specs=[
            pl.BlockSpec((_TB, _TOPK), lambda i: (jnp.maximum(i - 1, 0), 0)),
            pl.BlockSpec((_TB, _TOPK), lambda i: (jnp.maximum(i - 1, 0), 0)),
        ],
        out_shape=[
            jax.ShapeDtypeStruct((_TOKENS, _TOPK), jnp.int32),
            jax.ShapeDtypeStruct((_TOKENS, _TOPK), jnp.float32),
        ],
        scratch_shapes=[
            pltpu.VMEM((_TB, _EXPERTS), jnp.float32),
            pltpu.VMEM((_TB, _EXPERTS), jnp.float32),
        ],
    )(hs, wt, bias)
    return idxs, ws
